# baseline (speedup 1.0000x reference)
import functools

import jax
import jax.numpy as jnp
from jax import lax
from jax.experimental import pallas as pl
from jax.experimental.pallas import tpu as pltpu
from jax.experimental.pallas import tpu_sc as plsc

H = 2048; S = 2048; NH = 32; NKV = 4; HD = 64; E = 8; TOPK = 2; F = 1024
ROUTED_SCALE = 2.5; EPS = 1e-06; THETA = 8000000.0
BM = 256
NP = S * TOPK
L = NP + E * BM
NBLK = L // BM
BQ = 256

_bf16 = jnp.bfloat16
_f32 = jnp.float32


def _rmsnorm(x, w):
    xf = x.astype(_f32)
    v = jnp.mean(jnp.square(xf), axis=-1, keepdims=True)
    return (xf * lax.rsqrt(v + EPS)) * w


def _rope(x, pos):
    hd = x.shape[-1]
    inv = 1.0 / (THETA ** (jnp.arange(0, hd // 2, dtype=_f32) * 2.0 / hd))
    ang = pos.astype(_f32)[:, None] * inv[None, :]
    cos = jnp.concatenate([jnp.cos(ang), jnp.cos(ang)], axis=-1)[:, None, :]
    sin = jnp.concatenate([jnp.sin(ang), jnp.sin(ang)], axis=-1)[:, None, :]
    x1, x2 = jnp.split(x, 2, axis=-1)
    rot = jnp.concatenate([-x2, x1], axis=-1)
    return x * cos + rot * sin


def _qkv_kernel(x_ref, w_ref, o_ref):
    o_ref[...] = jnp.dot(x_ref[...], w_ref[...], preferred_element_type=_f32)


def _qkv_call(hb, w_qkv_b):
    n_out = (NH + 2 * NKV) * HD
    return pl.pallas_call(
        _qkv_kernel,
        grid=(8, n_out // 512),
        in_specs=[
            pl.BlockSpec((S // 8, H), lambda i, j: (i, 0)),
            pl.BlockSpec((H, 512), lambda i, j: (0, j)),
        ],
        out_specs=pl.BlockSpec((S // 8, 512), lambda i, j: (i, j)),
        out_shape=jax.ShapeDtypeStruct((S, n_out), _f32),
    )(hb, w_qkv_b)


def _attn_kernel(q_ref, k_ref, v_ref, o_ref, scr, m_scr, d_scr, o_scr):
    i = pl.program_id(1)
    nj = S // BQ
    tri = (lax.broadcasted_iota(jnp.int32, (BQ, BQ), 1)
           <= lax.broadcasted_iota(jnp.int32, (BQ, BQ), 0))
    for hh in range(2):
        q = q_ref[hh]
        for j in range(nj):
            ds = pl.ds(j * BQ, BQ)

            @pl.when(j < i)
            def _():
                kj = k_ref[0, ds, :]
                s = lax.dot_general(q, kj, (((1,), (1,)), ((), ())),
                                    preferred_element_type=_f32)
                scr[:, ds] = s
                mj = jnp.max(s, axis=-1, keepdims=True)
                if j == 0:
                    m_scr[...] = mj
                else:
                    m_scr[...] = jnp.maximum(m_scr[...], mj)

            @pl.when(j == i)
            def _():
                kj = k_ref[0, ds, :]
                s = lax.dot_general(q, kj, (((1,), (1,)), ((), ())),
                                    preferred_element_type=_f32)
                s = jnp.where(tri, s, _f32(-1e9))
                scr[:, ds] = s
                mj = jnp.max(s, axis=-1, keepdims=True)
                if j == 0:
                    m_scr[...] = mj
                else:
                    m_scr[...] = jnp.maximum(m_scr[...], mj)

        m = m_scr[...]
        for j in range(nj):
            ds = pl.ds(j * BQ, BQ)

            @pl.when(j <= i)
            def _():
                e = jnp.exp(scr[:, ds] - m)
                scr[:, ds] = e
                dj = jnp.sum(e, axis=-1, keepdims=True)
                if j == 0:
                    d_scr[...] = dj
                else:
                    d_scr[...] = d_scr[...] + dj

        d = d_scr[...]
        for j in range(nj):
            ds = pl.ds(j * BQ, BQ)

            @pl.when(j <= i)
            def _():
                p = (scr[:, ds] / d).astype(_bf16)
                pv = jnp.dot(p, v_ref[0, ds, :], preferred_element_type=_f32)
                if j == 0:
                    o_scr[...] = pv
                else:
                    o_scr[...] = o_scr[...] + pv

        o_ref[:, hh * HD : (hh + 1) * HD] = o_scr[...]


def _attn_call(qb3, kb3, vb3):
    return pl.pallas_call(
        _attn_kernel,
        grid=(NH // 2, S // BQ),
        in_specs=[
            pl.BlockSpec((2, BQ, HD), lambda h, i: (h, i, 0)),
            pl.BlockSpec((1, S, HD), lambda h, i: (h // 4, 0, 0)),
            pl.BlockSpec((1, S, HD), lambda h, i: (h // 4, 0, 0)),
        ],
        out_specs=pl.BlockSpec((BQ, 2 * HD), lambda h, i: (i, h)),
        out_shape=jax.ShapeDtypeStruct((S, NH * HD), _f32),
        scratch_shapes=[
            pltpu.VMEM((BQ, S), _f32),
            pltpu.VMEM((BQ, 1), _f32),
            pltpu.VMEM((BQ, 1), _f32),
            pltpu.VMEM((BQ, HD), _f32),
        ],
    )(qb3, kb3, vb3)


def _oproj_kernel(a_ref, hs_ref, wo_ref, h1_ref):
    o = jnp.dot(a_ref[...].astype(_bf16), wo_ref[...],
                preferred_element_type=_f32)
    h1_ref[...] = hs_ref[...] + o


def _oproj_call(attn, hs, wo_b):
    return pl.pallas_call(
        _oproj_kernel,
        grid=(8,),
        in_specs=[
            pl.BlockSpec((S // 8, NH * HD), lambda i: (i, 0)),
            pl.BlockSpec((S // 8, H), lambda i: (i, 0)),
            pl.BlockSpec((NH * HD, H), lambda i: (0, 0)),
        ],
        out_specs=pl.BlockSpec((S // 8, H), lambda i: (i, 0)),
        out_shape=jax.ShapeDtypeStruct((S, H), _f32),
    )(attn, hs, wo_b)


def _logits_kernel(x_ref, w_ref, o_ref):
    o_ref[...] = jnp.dot(x_ref[...], w_ref[...], preferred_element_type=_f32)


def _logits_call(x2b, wr_pad_b):
    return pl.pallas_call(
        _logits_kernel,
        grid=(8,),
        in_specs=[
            pl.BlockSpec((S // 8, H), lambda i: (i, 0)),
            pl.BlockSpec((H, 128), lambda i: (0, 0)),
        ],
        out_specs=pl.BlockSpec((S // 8, 128), lambda i: (i, 0)),
        out_shape=jax.ShapeDtypeStruct((S, 128), _f32),
    )(x2b, wr_pad_b)


def _router_kernel(lg_ref, bias_ref, slot_ref, wts_ref, bexp_ref):
    lg = lg_ref[...]
    sig = jax.nn.sigmoid(lg)
    bias = bias_ref[...]
    lane = lax.broadcasted_iota(jnp.int32, (16, 128, E), 2)
    biased = sig + bias

    m0 = jnp.max(biased, axis=-1, keepdims=True)
    is0 = biased == m0
    idx0 = jnp.min(jnp.where(is0, lane, E + 1), axis=-1, keepdims=True)
    sel0 = lane == idx0
    w0 = jnp.sum(jnp.where(sel0, sig, 0.0), axis=-1, keepdims=True)

    b1 = jnp.where(sel0, _f32(-1e30), biased)
    m1 = jnp.max(b1, axis=-1, keepdims=True)
    is1 = b1 == m1
    idx1 = jnp.min(jnp.where(is1, lane, E + 1), axis=-1, keepdims=True)
    sel1 = lane == idx1
    w1 = jnp.sum(jnp.where(sel1, sig, 0.0), axis=-1, keepdims=True)

    ssum = w0 + w1
    w0 = (w0 / ssum) * ROUTED_SCALE
    w1 = (w1 / ssum) * ROUTED_SCALE

    e_pair = jnp.concatenate([idx0[:, :, 0], idx1[:, :, 0]], axis=0)
    w_pair = jnp.concatenate([w0[:, :, 0], w1[:, :, 0]], axis=0)

    r_iota = lax.broadcasted_iota(jnp.int32, (128, 128), 0)
    c_iota = lax.broadcasted_iota(jnp.int32, (128, 128), 1)
    u128 = (r_iota <= c_iota).astype(_bf16)
    r32 = lax.broadcasted_iota(jnp.int32, (32, 32), 0)
    c32 = lax.broadcasted_iota(jnp.int32, (32, 32), 1)
    lstrict = (r32 > c32).astype(_bf16)

    slotf = jnp.zeros((32, 128), _f32)
    start = jnp.int32(0)
    starts = []
    for e in range(E):
        mk = (e_pair == e).astype(_f32)
        incl = jnp.dot(mk.astype(_bf16), u128, preferred_element_type=_f32)
        rowtot = jnp.sum(mk, axis=-1, keepdims=True)
        rowoff = jnp.dot(lstrict, rowtot.astype(_bf16),
                         preferred_element_type=_f32)
        rank = incl - mk + rowoff
        starts.append(start)
        slotf = slotf + mk * (start.astype(_f32) + rank)
        cnt = jnp.sum(mk).astype(jnp.int32)
        pc = ((cnt + BM - 1) // BM) * BM
        start = start + pc

    slot_ref[...] = slotf.astype(jnp.int32)
    wts_ref[...] = w_pair

    blk = lax.broadcasted_iota(jnp.int32, (1, 128), 1)
    bexp = jnp.zeros((1, 128), jnp.int32)
    for e in range(E):
        bexp = bexp + jnp.where(blk >= starts[e] // BM, 1, 0)
    bexp_ref[...] = bexp - 1


def _router_call(lg3, bias_pad):
    return pl.pallas_call(
        _router_kernel,
        grid=(1,),
        in_specs=[
            pl.BlockSpec((16, 128, E), lambda i: (0, 0, 0)),
            pl.BlockSpec((1, 1, E), lambda i: (0, 0, 0)),
        ],
        out_specs=[
            pl.BlockSpec((32, 128), lambda i: (0, 0)),
            pl.BlockSpec((32, 128), lambda i: (0, 0)),
            pl.BlockSpec((1, 128), lambda i: (0, 0)),
        ],
        out_shape=[
            jax.ShapeDtypeStruct((32, 128), jnp.int32),
            jax.ShapeDtypeStruct((32, 128), _f32),
            jax.ShapeDtypeStruct((1, 128), jnp.int32),
        ],
    )(lg3, bias_pad)


def _moe_kernel(bexp_ref, xs_ref, wg_ref, wu_ref, wd_ref, ys_ref):
    x = xs_ref[...].astype(_bf16)
    g = jnp.dot(x, wg_ref[0], preferred_element_type=_f32)
    u = jnp.dot(x, wu_ref[0], preferred_element_type=_f32)
    act = (g * jax.nn.sigmoid(g)) * u
    ys_ref[...] = jnp.dot(act.astype(_bf16), wd_ref[0],
                          preferred_element_type=_f32)


def _moe_call(bexp, xs, wg_b, wu_b, wd_b):
    grid_spec = pltpu.PrefetchScalarGridSpec(
        num_scalar_prefetch=1,
        grid=(NBLK,),
        in_specs=[
            pl.BlockSpec((BM, H), lambda b, bexp: (b, 0)),
            pl.BlockSpec((1, H, F), lambda b, bexp: (bexp[b], 0, 0)),
            pl.BlockSpec((1, H, F), lambda b, bexp: (bexp[b], 0, 0)),
            pl.BlockSpec((1, F, H), lambda b, bexp: (bexp[b], 0, 0)),
        ],
        out_specs=pl.BlockSpec((BM, H), lambda b, bexp: (b, 0)),
    )
    return pl.pallas_call(
        _moe_kernel,
        grid_spec=grid_spec,
        out_shape=jax.ShapeDtypeStruct((L, H), _f32),
    )(bexp, xs, wg_b, wu_b, wd_b)


def _dispatch_body(x2_hbm, slot_hbm, xs_hbm, idx_v, rows_v, sem):
    wid = lax.axis_index("s") * 2 + lax.axis_index("c")
    pltpu.sync_copy(slot_hbm.at[wid], idx_v)
    t0 = (wid % 16) * 128
    for j in range(8):
        pltpu.sync_copy(x2_hbm.at[pl.ds(t0 + j * 16, 16)], rows_v)
        pltpu.async_copy(rows_v, xs_hbm.at[idx_v.at[j]], sem).wait()


def _dispatch_call(x2, slot3):
    mesh = plsc.VectorSubcoreMesh(core_axis_name="c", subcore_axis_name="s")
    fn = functools.partial(
        pl.kernel,
        out_type=jax.ShapeDtypeStruct((L, H), _f32),
        mesh=mesh,
        scratch_types=[
            pltpu.VMEM((8, 16), jnp.int32),
            pltpu.VMEM((16, H), _f32),
            pltpu.SemaphoreType.DMA,
        ],
    )(_dispatch_body)
    return fn(x2, slot3)


def _combine_body(ys_hbm, slot_hbm, wts_hbm, h1_hbm, out_hbm,
                  idx0_v, idx1_v, w0_v, w1_v, rows0_v, rows1_v, h1_v, out_v,
                  sem):
    wid = lax.axis_index("s") * 2 + lax.axis_index("c")
    r0 = wid // 2
    half = 4 * (wid % 2)
    pltpu.sync_copy(slot_hbm.at[r0, pl.ds(half, 4)], idx0_v)
    pltpu.sync_copy(slot_hbm.at[16 + r0, pl.ds(half, 4)], idx1_v)
    pltpu.sync_copy(wts_hbm.at[r0, pl.ds(half, 4)], w0_v)
    pltpu.sync_copy(wts_hbm.at[16 + r0, pl.ds(half, 4)], w1_v)
    tok0 = wid * 64
    for c in range(8):
        tb = tok0 + c * 8
        pltpu.sync_copy(h1_hbm.at[pl.ds(tb, 8)], h1_v)
        i0 = idx0_v.at[c // 2, pl.ds(8 * (c % 2), 8)]
        i1 = idx1_v.at[c // 2, pl.ds(8 * (c % 2), 8)]
        pltpu.async_copy(ys_hbm.at[i0], rows0_v, sem).wait()
        pltpu.async_copy(ys_hbm.at[i1], rows1_v, sem).wait()
        w0row = w0_v[c // 2, :]
        w1row = w1_v[c // 2, :]
        for j in range(8):
            w0s = w0row[8 * (c % 2) + j]
            w1s = w1row[8 * (c % 2) + j]

            def body(i, _):
                sl = pl.ds(i * 16, 16)
                out_v[j, sl] = (h1_v[j, sl] + w0s * rows0_v[j, sl]
                                + w1s * rows1_v[j, sl])
                return 0

            lax.fori_loop(0, H // 16, body, 0)
        pltpu.sync_copy(out_v, out_hbm.at[pl.ds(tb, 8)])


def _combine_call(ys, slot3, wts3, h1):
    mesh = plsc.VectorSubcoreMesh(core_axis_name="c", subcore_axis_name="s")
    fn = functools.partial(
        pl.kernel,
        out_type=jax.ShapeDtypeStruct((S, H), _f32),
        mesh=mesh,
        scratch_types=[
            pltpu.VMEM((4, 16), jnp.int32),
            pltpu.VMEM((4, 16), jnp.int32),
            pltpu.VMEM((4, 16), _f32),
            pltpu.VMEM((4, 16), _f32),
            pltpu.VMEM((8, H), _f32),
            pltpu.VMEM((8, H), _f32),
            pltpu.VMEM((8, H), _f32),
            pltpu.VMEM((8, H), _f32),
            pltpu.SemaphoreType.DMA,
        ],
    )(_combine_body)
    return fn(ys, slot3, wts3, h1)


def kernel(hidden_states, ln1_w, ln2_w, w_qkv, w_o, q_norm_w, k_norm_w,
           router_w, expert_bias, w_gate, w_up, w_down):
    hs2 = hidden_states.reshape(S, H)
    h = _rmsnorm(hidden_states, ln1_w).reshape(S, H)
    qkv = _qkv_call(h.astype(_bf16), w_qkv.astype(_bf16))

    q = qkv[:, : NH * HD].reshape(1, S, NH, HD)
    k = qkv[:, NH * HD : (NH + NKV) * HD].reshape(1, S, NKV, HD)
    v = qkv[:, (NH + NKV) * HD :].reshape(1, S, NKV, HD)
    pos = jnp.arange(S)
    qr = _rope(_rmsnorm(q, q_norm_w), pos)[0]
    kr = _rope(_rmsnorm(k, k_norm_w), pos)[0]
    qb3 = (qr * (1.0 / 8.0)).astype(_bf16).transpose(1, 0, 2)
    kb3 = kr.astype(_bf16).transpose(1, 0, 2)
    vb3 = v[0].astype(_bf16).transpose(1, 0, 2)
    attn = _attn_call(qb3, kb3, vb3)

    h1 = _oproj_call(attn, hs2, w_o.astype(_bf16))
    x2 = _rmsnorm(h1, ln2_w)
    wr_pad = jnp.pad(router_w, ((0, 0), (0, 128 - E))).astype(_bf16)
    lg = _logits_call(x2.astype(_bf16), wr_pad)

    lg3 = lg[:, :E].reshape(16, 128, E)
    slot, wts, bexp = _router_call(lg3, expert_bias.reshape(1, 1, E))
    bexp_s = bexp.reshape(-1)[:NBLK]
    slot3 = slot.reshape(32, 8, 16)
    wts3 = wts.reshape(32, 8, 16)

    xs = _dispatch_call(x2, slot3)
    ys = _moe_call(bexp_s, xs, w_gate.astype(_bf16), w_up.astype(_bf16),
                   w_down.astype(_bf16))
    out = _combine_call(ys, slot3, wts3, h1)
    return out.reshape(1, S, H)


# speedup vs baseline: 1.5275x; 1.5275x over previous
import functools

import jax
import jax.numpy as jnp
from jax import lax
from jax.experimental import pallas as pl
from jax.experimental.pallas import tpu as pltpu
from jax.experimental.pallas import tpu_sc as plsc

H = 2048; S = 2048; NH = 32; NKV = 4; HD = 64; E = 8; TOPK = 2; F = 1024
ROUTED_SCALE = 2.5; EPS = 1e-06; THETA = 8000000.0
BM = 128
NP = S * TOPK
L = NP + E * BM
NBLK = L // BM
BQ = 256

_bf16 = jnp.bfloat16
_f32 = jnp.float32


def _rmsnorm(x, w):
    xf = x.astype(_f32)
    v = jnp.mean(jnp.square(xf), axis=-1, keepdims=True)
    return (xf * lax.rsqrt(v + EPS)) * w


def _rope(x, pos):
    hd = x.shape[-1]
    inv = 1.0 / (THETA ** (jnp.arange(0, hd // 2, dtype=_f32) * 2.0 / hd))
    ang = pos.astype(_f32)[:, None] * inv[None, :]
    cos = jnp.concatenate([jnp.cos(ang), jnp.cos(ang)], axis=-1)[:, None, :]
    sin = jnp.concatenate([jnp.sin(ang), jnp.sin(ang)], axis=-1)[:, None, :]
    x1, x2 = jnp.split(x, 2, axis=-1)
    rot = jnp.concatenate([-x2, x1], axis=-1)
    return x * cos + rot * sin


def _qkv_kernel(x_ref, w_ref, o_ref):
    o_ref[...] = jnp.dot(x_ref[...], w_ref[...], preferred_element_type=_f32)


def _qkv_call(hb, w_qkv_b):
    n_out = (NH + 2 * NKV) * HD
    return pl.pallas_call(
        _qkv_kernel,
        grid=(8, n_out // 512),
        in_specs=[
            pl.BlockSpec((S // 8, H), lambda i, j: (i, 0)),
            pl.BlockSpec((H, 512), lambda i, j: (0, j)),
        ],
        out_specs=pl.BlockSpec((S // 8, 512), lambda i, j: (i, j)),
        out_shape=jax.ShapeDtypeStruct((S, n_out), _f32),
    )(hb, w_qkv_b)


def _attn_kernel(q_ref, k_ref, v_ref, o_ref):
    i = pl.program_id(1)
    kb = k_ref[0]
    vb = v_ref[0]
    row = i * BQ + lax.broadcasted_iota(jnp.int32, (BQ, S), 0)
    colj = lax.broadcasted_iota(jnp.int32, (BQ, S), 1)
    causal = colj <= row
    outs = []
    for hh in range(2):
        s = lax.dot_general(q_ref[hh], kb, (((1,), (1,)), ((), ())),
                            preferred_element_type=_f32)
        s = jnp.where(causal, s, _f32(-1e9))
        m = jnp.max(s, axis=-1, keepdims=True)
        p = jnp.exp(s - m)
        p = p / jnp.sum(p, axis=-1, keepdims=True)
        outs.append(jnp.dot(p.astype(_bf16), vb,
                            preferred_element_type=_f32))
    o_ref[...] = jnp.concatenate(outs, axis=1)


def _attn_call(qb3, kb3, vb3):
    return pl.pallas_call(
        _attn_kernel,
        grid=(NH // 2, S // BQ),
        in_specs=[
            pl.BlockSpec((2, BQ, HD), lambda h, i: (h, i, 0)),
            pl.BlockSpec((1, S, HD), lambda h, i: (h // 4, 0, 0)),
            pl.BlockSpec((1, S, HD), lambda h, i: (h // 4, 0, 0)),
        ],
        out_specs=pl.BlockSpec((BQ, 2 * HD), lambda h, i: (i, h)),
        out_shape=jax.ShapeDtypeStruct((S, NH * HD), _f32),
    )(qb3, kb3, vb3)


def _oproj_kernel(a_ref, hs_ref, wo_ref, h1_ref):
    o = jnp.dot(a_ref[...].astype(_bf16), wo_ref[...],
                preferred_element_type=_f32)
    h1_ref[...] = hs_ref[...] + o


def _oproj_call(attn, hs, wo_b):
    return pl.pallas_call(
        _oproj_kernel,
        grid=(8,),
        in_specs=[
            pl.BlockSpec((S // 8, NH * HD), lambda i: (i, 0)),
            pl.BlockSpec((S // 8, H), lambda i: (i, 0)),
            pl.BlockSpec((NH * HD, H), lambda i: (0, 0)),
        ],
        out_specs=pl.BlockSpec((S // 8, H), lambda i: (i, 0)),
        out_shape=jax.ShapeDtypeStruct((S, H), _f32),
    )(attn, hs, wo_b)


def _logits_kernel(x_ref, w_ref, o_ref):
    o_ref[...] = jnp.dot(x_ref[...], w_ref[...], preferred_element_type=_f32)


def _logits_call(x2b, wr_pad_b):
    return pl.pallas_call(
        _logits_kernel,
        grid=(8,),
        in_specs=[
            pl.BlockSpec((S // 8, H), lambda i: (i, 0)),
            pl.BlockSpec((H, 128), lambda i: (0, 0)),
        ],
        out_specs=pl.BlockSpec((S // 8, 128), lambda i: (i, 0)),
        out_shape=jax.ShapeDtypeStruct((S, 128), _f32),
    )(x2b, wr_pad_b)


def _router_kernel(lg_ref, bias_ref, slot_ref, wts_ref, bexp_ref):
    lg = lg_ref[...]
    sig = jax.nn.sigmoid(lg)
    bias = bias_ref[...]
    lane = lax.broadcasted_iota(jnp.int32, (16, 128, E), 2)
    biased = sig + bias

    m0 = jnp.max(biased, axis=-1, keepdims=True)
    is0 = biased == m0
    idx0 = jnp.min(jnp.where(is0, lane, E + 1), axis=-1, keepdims=True)
    sel0 = lane == idx0
    w0 = jnp.sum(jnp.where(sel0, sig, 0.0), axis=-1, keepdims=True)

    b1 = jnp.where(sel0, _f32(-1e30), biased)
    m1 = jnp.max(b1, axis=-1, keepdims=True)
    is1 = b1 == m1
    idx1 = jnp.min(jnp.where(is1, lane, E + 1), axis=-1, keepdims=True)
    sel1 = lane == idx1
    w1 = jnp.sum(jnp.where(sel1, sig, 0.0), axis=-1, keepdims=True)

    ssum = w0 + w1
    w0 = (w0 / ssum) * ROUTED_SCALE
    w1 = (w1 / ssum) * ROUTED_SCALE

    e_pair = jnp.concatenate([idx0[:, :, 0], idx1[:, :, 0]], axis=0)
    w_pair = jnp.concatenate([w0[:, :, 0], w1[:, :, 0]], axis=0)

    r_iota = lax.broadcasted_iota(jnp.int32, (128, 128), 0)
    c_iota = lax.broadcasted_iota(jnp.int32, (128, 128), 1)
    u128 = (r_iota <= c_iota).astype(_bf16)
    r32 = lax.broadcasted_iota(jnp.int32, (32, 32), 0)
    c32 = lax.broadcasted_iota(jnp.int32, (32, 32), 1)
    lstrict = (r32 > c32).astype(_bf16)

    slotf = jnp.zeros((32, 128), _f32)
    start = jnp.int32(0)
    starts = []
    for e in range(E):
        mk = (e_pair == e).astype(_f32)
        incl = jnp.dot(mk.astype(_bf16), u128, preferred_element_type=_f32)
        rowtot = jnp.sum(mk, axis=-1, keepdims=True)
        rowoff = jnp.dot(lstrict, rowtot.astype(_bf16),
                         preferred_element_type=_f32)
        rank = incl - mk + rowoff
        starts.append(start)
        slotf = slotf + mk * (start.astype(_f32) + rank)
        cnt = jnp.sum(mk).astype(jnp.int32)
        pc = ((cnt + BM - 1) // BM) * BM
        start = start + pc

    slot_ref[...] = slotf.astype(jnp.int32)
    wts_ref[...] = w_pair

    blk = lax.broadcasted_iota(jnp.int32, (1, 128), 1)
    bexp = jnp.zeros((1, 128), jnp.int32)
    for e in range(E):
        bexp = bexp + jnp.where(blk >= starts[e] // BM, 1, 0)
    bexp_ref[...] = bexp - 1


def _router_call(lg3, bias_pad):
    return pl.pallas_call(
        _router_kernel,
        grid=(1,),
        in_specs=[
            pl.BlockSpec((16, 128, E), lambda i: (0, 0, 0)),
            pl.BlockSpec((1, 1, E), lambda i: (0, 0, 0)),
        ],
        out_specs=[
            pl.BlockSpec((32, 128), lambda i: (0, 0)),
            pl.BlockSpec((32, 128), lambda i: (0, 0)),
            pl.BlockSpec((1, 128), lambda i: (0, 0)),
        ],
        out_shape=[
            jax.ShapeDtypeStruct((32, 128), jnp.int32),
            jax.ShapeDtypeStruct((32, 128), _f32),
            jax.ShapeDtypeStruct((1, 128), jnp.int32),
        ],
    )(lg3, bias_pad)


def _moe_kernel(bexp_ref, xs_ref, wg_ref, wu_ref, wd_ref, ys_ref):
    x = xs_ref[...].astype(_bf16)
    g = jnp.dot(x, wg_ref[0], preferred_element_type=_f32)
    u = jnp.dot(x, wu_ref[0], preferred_element_type=_f32)
    act = (g * jax.nn.sigmoid(g)) * u
    ys_ref[...] = jnp.dot(act.astype(_bf16), wd_ref[0],
                          preferred_element_type=_f32)


def _moe_call(bexp, xs, wg_b, wu_b, wd_b):
    grid_spec = pltpu.PrefetchScalarGridSpec(
        num_scalar_prefetch=1,
        grid=(NBLK,),
        in_specs=[
            pl.BlockSpec((BM, H), lambda b, bexp: (b, 0)),
            pl.BlockSpec((1, H, F), lambda b, bexp: (bexp[b], 0, 0)),
            pl.BlockSpec((1, H, F), lambda b, bexp: (bexp[b], 0, 0)),
            pl.BlockSpec((1, F, H), lambda b, bexp: (bexp[b], 0, 0)),
        ],
        out_specs=pl.BlockSpec((BM, H), lambda b, bexp: (b, 0)),
    )
    return pl.pallas_call(
        _moe_kernel,
        grid_spec=grid_spec,
        out_shape=jax.ShapeDtypeStruct((L, H), _f32),
    )(bexp, xs, wg_b, wu_b, wd_b)


def _dispatch_body(x2_hbm, slot_hbm, xs_hbm, idx_v, rows_v, sem):
    wid = lax.axis_index("s") * 2 + lax.axis_index("c")
    pltpu.sync_copy(slot_hbm.at[wid], idx_v)
    t0 = (wid % 16) * 128
    for j in range(8):
        pltpu.sync_copy(x2_hbm.at[pl.ds(t0 + j * 16, 16)], rows_v)
        pltpu.async_copy(rows_v, xs_hbm.at[idx_v.at[j]], sem).wait()


def _dispatch_call(x2, slot3):
    mesh = plsc.VectorSubcoreMesh(core_axis_name="c", subcore_axis_name="s")
    fn = functools.partial(
        pl.kernel,
        out_type=jax.ShapeDtypeStruct((L, H), _f32),
        mesh=mesh,
        scratch_types=[
            pltpu.VMEM((8, 16), jnp.int32),
            pltpu.VMEM((16, H), _f32),
            pltpu.SemaphoreType.DMA,
        ],
    )(_dispatch_body)
    return fn(x2, slot3)


def _combine_body(ys_hbm, slot_hbm, wts_hbm, h1_hbm, out_hbm,
                  idx0_v, idx1_v, w0_v, w1_v, rows0_v, rows1_v, h1_v, out_v,
                  sem):
    wid = lax.axis_index("s") * 2 + lax.axis_index("c")
    r0 = wid // 2
    half = 4 * (wid % 2)
    pltpu.sync_copy(slot_hbm.at[r0, pl.ds(half, 4)], idx0_v)
    pltpu.sync_copy(slot_hbm.at[16 + r0, pl.ds(half, 4)], idx1_v)
    pltpu.sync_copy(wts_hbm.at[r0, pl.ds(half, 4)], w0_v)
    pltpu.sync_copy(wts_hbm.at[16 + r0, pl.ds(half, 4)], w1_v)
    tok0 = wid * 64
    for c in range(8):
        tb = tok0 + c * 8
        pltpu.sync_copy(h1_hbm.at[pl.ds(tb, 8)], h1_v)
        i0 = idx0_v.at[c // 2, pl.ds(8 * (c % 2), 8)]
        i1 = idx1_v.at[c // 2, pl.ds(8 * (c % 2), 8)]
        pltpu.async_copy(ys_hbm.at[i0], rows0_v, sem).wait()
        pltpu.async_copy(ys_hbm.at[i1], rows1_v, sem).wait()
        w0row = w0_v[c // 2, :]
        w1row = w1_v[c // 2, :]
        for j in range(8):
            w0s = w0row[8 * (c % 2) + j]
            w1s = w1row[8 * (c % 2) + j]

            def body(i, _):
                sl = pl.ds(i * 16, 16)
                out_v[j, sl] = (h1_v[j, sl] + w0s * rows0_v[j, sl]
                                + w1s * rows1_v[j, sl])
                return 0

            lax.fori_loop(0, H // 16, body, 0)
        pltpu.sync_copy(out_v, out_hbm.at[pl.ds(tb, 8)])


def _combine_call(ys, slot3, wts3, h1):
    mesh = plsc.VectorSubcoreMesh(core_axis_name="c", subcore_axis_name="s")
    fn = functools.partial(
        pl.kernel,
        out_type=jax.ShapeDtypeStruct((S, H), _f32),
        mesh=mesh,
        scratch_types=[
            pltpu.VMEM((4, 16), jnp.int32),
            pltpu.VMEM((4, 16), jnp.int32),
            pltpu.VMEM((4, 16), _f32),
            pltpu.VMEM((4, 16), _f32),
            pltpu.VMEM((8, H), _f32),
            pltpu.VMEM((8, H), _f32),
            pltpu.VMEM((8, H), _f32),
            pltpu.VMEM((8, H), _f32),
            pltpu.SemaphoreType.DMA,
        ],
    )(_combine_body)
    return fn(ys, slot3, wts3, h1)


def kernel(hidden_states, ln1_w, ln2_w, w_qkv, w_o, q_norm_w, k_norm_w,
           router_w, expert_bias, w_gate, w_up, w_down):
    hs2 = hidden_states.reshape(S, H)
    h = _rmsnorm(hidden_states, ln1_w).reshape(S, H)
    qkv = _qkv_call(h.astype(_bf16), w_qkv.astype(_bf16))

    q = qkv[:, : NH * HD].reshape(1, S, NH, HD)
    k = qkv[:, NH * HD : (NH + NKV) * HD].reshape(1, S, NKV, HD)
    v = qkv[:, (NH + NKV) * HD :].reshape(1, S, NKV, HD)
    pos = jnp.arange(S)
    qr = _rope(_rmsnorm(q, q_norm_w), pos)[0]
    kr = _rope(_rmsnorm(k, k_norm_w), pos)[0]
    qb3 = (qr * (1.0 / 8.0)).astype(_bf16).transpose(1, 0, 2)
    kb3 = kr.astype(_bf16).transpose(1, 0, 2)
    vb3 = v[0].astype(_bf16).transpose(1, 0, 2)
    attn = _attn_call(qb3, kb3, vb3)

    h1 = _oproj_call(attn, hs2, w_o.astype(_bf16))
    x2 = _rmsnorm(h1, ln2_w)
    wr_pad = jnp.pad(router_w, ((0, 0), (0, 128 - E))).astype(_bf16)
    lg = _logits_call(x2.astype(_bf16), wr_pad)

    lg3 = lg[:, :E].reshape(16, 128, E)
    slot, wts, bexp = _router_call(lg3, expert_bias.reshape(1, 1, E))
    bexp_s = bexp.reshape(-1)[:NBLK]
    slot3 = slot.reshape(32, 8, 16)
    wts3 = wts.reshape(32, 8, 16)

    xs = _dispatch_call(x2, slot3)
    ys = _moe_call(bexp_s, xs, w_gate.astype(_bf16), w_up.astype(_bf16),
                   w_down.astype(_bf16))
    out = _combine_call(ys, slot3, wts3, h1)
    return out.reshape(1, S, H)


# speedup vs baseline: 1.5418x; 1.0094x over previous
import functools

import jax
import jax.numpy as jnp
from jax import lax
from jax.experimental import pallas as pl
from jax.experimental.pallas import tpu as pltpu
from jax.experimental.pallas import tpu_sc as plsc

H = 2048; S = 2048; NH = 32; NKV = 4; HD = 64; E = 8; TOPK = 2; F = 1024
ROUTED_SCALE = 2.5; EPS = 1e-06; THETA = 8000000.0
BM = 128
NP = S * TOPK
L = NP + E * BM
NBLK = L // BM
BQ = 512

_bf16 = jnp.bfloat16
_f32 = jnp.float32


def _rmsnorm(x, w):
    xf = x.astype(_f32)
    v = jnp.mean(jnp.square(xf), axis=-1, keepdims=True)
    return (xf * lax.rsqrt(v + EPS)) * w


def _rope(x, pos):
    hd = x.shape[-1]
    inv = 1.0 / (THETA ** (jnp.arange(0, hd // 2, dtype=_f32) * 2.0 / hd))
    ang = pos.astype(_f32)[:, None] * inv[None, :]
    cos = jnp.concatenate([jnp.cos(ang), jnp.cos(ang)], axis=-1)[:, None, :]
    sin = jnp.concatenate([jnp.sin(ang), jnp.sin(ang)], axis=-1)[:, None, :]
    x1, x2 = jnp.split(x, 2, axis=-1)
    rot = jnp.concatenate([-x2, x1], axis=-1)
    return x * cos + rot * sin


def _qkv_kernel(x_ref, w_ref, o_ref):
    o_ref[...] = jnp.dot(x_ref[...], w_ref[...], preferred_element_type=_f32)


def _qkv_call(hb, w_qkv_b):
    n_out = (NH + 2 * NKV) * HD
    return pl.pallas_call(
        _qkv_kernel,
        grid=(8, n_out // 512),
        in_specs=[
            pl.BlockSpec((S // 8, H), lambda i, j: (i, 0)),
            pl.BlockSpec((H, 512), lambda i, j: (0, j)),
        ],
        out_specs=pl.BlockSpec((S // 8, 512), lambda i, j: (i, j)),
        out_shape=jax.ShapeDtypeStruct((S, n_out), _f32),
    )(hb, w_qkv_b)


def _attn_kernel(q_ref, k_ref, v_ref, o_ref):
    i = pl.program_id(1)
    kb = k_ref[0]
    vb = v_ref[0]
    row = i * BQ + lax.broadcasted_iota(jnp.int32, (BQ, S), 0)
    colj = lax.broadcasted_iota(jnp.int32, (BQ, S), 1)
    causal = colj <= row
    outs = []
    for hh in range(2):
        s = lax.dot_general(q_ref[hh], kb, (((1,), (1,)), ((), ())),
                            preferred_element_type=_f32)
        s = jnp.where(causal, s, _f32(-1e9))
        m = jnp.max(s, axis=-1, keepdims=True)
        p = jnp.exp(s - m)
        p = p / jnp.sum(p, axis=-1, keepdims=True)
        outs.append(jnp.dot(p.astype(_bf16), vb,
                            preferred_element_type=_f32))
    o_ref[...] = jnp.concatenate(outs, axis=1)


def _attn_call(qb3, kb3, vb3):
    return pl.pallas_call(
        _attn_kernel,
        grid=(NH // 2, S // BQ),
        in_specs=[
            pl.BlockSpec((2, BQ, HD), lambda h, i: (h, i, 0)),
            pl.BlockSpec((1, S, HD), lambda h, i: (h // 4, 0, 0)),
            pl.BlockSpec((1, S, HD), lambda h, i: (h // 4, 0, 0)),
        ],
        out_specs=pl.BlockSpec((BQ, 2 * HD), lambda h, i: (i, h)),
        out_shape=jax.ShapeDtypeStruct((S, NH * HD), _f32),
    )(qb3, kb3, vb3)


def _oproj_kernel(a_ref, hs_ref, wo_ref, h1_ref):
    o = jnp.dot(a_ref[...].astype(_bf16), wo_ref[...],
                preferred_element_type=_f32)
    h1_ref[...] = hs_ref[...] + o


def _oproj_call(attn, hs, wo_b):
    return pl.pallas_call(
        _oproj_kernel,
        grid=(8,),
        in_specs=[
            pl.BlockSpec((S // 8, NH * HD), lambda i: (i, 0)),
            pl.BlockSpec((S // 8, H), lambda i: (i, 0)),
            pl.BlockSpec((NH * HD, H), lambda i: (0, 0)),
        ],
        out_specs=pl.BlockSpec((S // 8, H), lambda i: (i, 0)),
        out_shape=jax.ShapeDtypeStruct((S, H), _f32),
    )(attn, hs, wo_b)


def _logits_kernel(x_ref, w_ref, o_ref):
    o_ref[...] = jnp.dot(x_ref[...], w_ref[...], preferred_element_type=_f32)


def _logits_call(x2b, wr_pad_b):
    return pl.pallas_call(
        _logits_kernel,
        grid=(8,),
        in_specs=[
            pl.BlockSpec((S // 8, H), lambda i: (i, 0)),
            pl.BlockSpec((H, 128), lambda i: (0, 0)),
        ],
        out_specs=pl.BlockSpec((S // 8, 128), lambda i: (i, 0)),
        out_shape=jax.ShapeDtypeStruct((S, 128), _f32),
    )(x2b, wr_pad_b)


def _router_kernel(lg_ref, bias_ref, slot_ref, wts_ref, bexp_ref):
    lg = lg_ref[...]
    sig = jax.nn.sigmoid(lg)
    bias = bias_ref[...]
    lane = lax.broadcasted_iota(jnp.int32, (16, 128, E), 2)
    biased = sig + bias

    m0 = jnp.max(biased, axis=-1, keepdims=True)
    is0 = biased == m0
    idx0 = jnp.min(jnp.where(is0, lane, E + 1), axis=-1, keepdims=True)
    sel0 = lane == idx0
    w0 = jnp.sum(jnp.where(sel0, sig, 0.0), axis=-1, keepdims=True)

    b1 = jnp.where(sel0, _f32(-1e30), biased)
    m1 = jnp.max(b1, axis=-1, keepdims=True)
    is1 = b1 == m1
    idx1 = jnp.min(jnp.where(is1, lane, E + 1), axis=-1, keepdims=True)
    sel1 = lane == idx1
    w1 = jnp.sum(jnp.where(sel1, sig, 0.0), axis=-1, keepdims=True)

    ssum = w0 + w1
    w0 = (w0 / ssum) * ROUTED_SCALE
    w1 = (w1 / ssum) * ROUTED_SCALE

    e_pair = jnp.concatenate([idx0[:, :, 0], idx1[:, :, 0]], axis=0)
    w_pair = jnp.concatenate([w0[:, :, 0], w1[:, :, 0]], axis=0)

    r_iota = lax.broadcasted_iota(jnp.int32, (128, 128), 0)
    c_iota = lax.broadcasted_iota(jnp.int32, (128, 128), 1)
    u128 = (r_iota <= c_iota).astype(_bf16)
    r32 = lax.broadcasted_iota(jnp.int32, (32, 32), 0)
    c32 = lax.broadcasted_iota(jnp.int32, (32, 32), 1)
    lstrict = (r32 > c32).astype(_bf16)

    slotf = jnp.zeros((32, 128), _f32)
    start = jnp.int32(0)
    starts = []
    for e in range(E):
        mk = (e_pair == e).astype(_f32)
        incl = jnp.dot(mk.astype(_bf16), u128, preferred_element_type=_f32)
        rowtot = jnp.sum(mk, axis=-1, keepdims=True)
        rowoff = jnp.dot(lstrict, rowtot.astype(_bf16),
                         preferred_element_type=_f32)
        rank = incl - mk + rowoff
        starts.append(start)
        slotf = slotf + mk * (start.astype(_f32) + rank)
        cnt = jnp.sum(mk).astype(jnp.int32)
        pc = ((cnt + BM - 1) // BM) * BM
        start = start + pc

    slot_ref[...] = slotf.astype(jnp.int32)
    wts_ref[...] = w_pair

    blk = lax.broadcasted_iota(jnp.int32, (1, 128), 1)
    bexp = jnp.zeros((1, 128), jnp.int32)
    for e in range(E):
        bexp = bexp + jnp.where(blk >= starts[e] // BM, 1, 0)
    bexp_ref[...] = bexp - 1


def _router_call(lg3, bias_pad):
    return pl.pallas_call(
        _router_kernel,
        grid=(1,),
        in_specs=[
            pl.BlockSpec((16, 128, E), lambda i: (0, 0, 0)),
            pl.BlockSpec((1, 1, E), lambda i: (0, 0, 0)),
        ],
        out_specs=[
            pl.BlockSpec((32, 128), lambda i: (0, 0)),
            pl.BlockSpec((32, 128), lambda i: (0, 0)),
            pl.BlockSpec((1, 128), lambda i: (0, 0)),
        ],
        out_shape=[
            jax.ShapeDtypeStruct((32, 128), jnp.int32),
            jax.ShapeDtypeStruct((32, 128), _f32),
            jax.ShapeDtypeStruct((1, 128), jnp.int32),
        ],
    )(lg3, bias_pad)


def _moe_kernel(bexp_ref, xs_ref, wg_ref, wu_ref, wd_ref, ys_ref):
    x = xs_ref[...].astype(_bf16)
    g = jnp.dot(x, wg_ref[0], preferred_element_type=_f32)
    u = jnp.dot(x, wu_ref[0], preferred_element_type=_f32)
    act = (g * jax.nn.sigmoid(g)) * u
    ys_ref[...] = jnp.dot(act.astype(_bf16), wd_ref[0],
                          preferred_element_type=_f32)


def _moe_call(bexp, xs, wg_b, wu_b, wd_b):
    grid_spec = pltpu.PrefetchScalarGridSpec(
        num_scalar_prefetch=1,
        grid=(NBLK,),
        in_specs=[
            pl.BlockSpec((BM, H), lambda b, bexp: (b, 0)),
            pl.BlockSpec((1, H, F), lambda b, bexp: (bexp[b], 0, 0)),
            pl.BlockSpec((1, H, F), lambda b, bexp: (bexp[b], 0, 0)),
            pl.BlockSpec((1, F, H), lambda b, bexp: (bexp[b], 0, 0)),
        ],
        out_specs=pl.BlockSpec((BM, H), lambda b, bexp: (b, 0)),
    )
    return pl.pallas_call(
        _moe_kernel,
        grid_spec=grid_spec,
        out_shape=jax.ShapeDtypeStruct((L, H), _f32),
    )(bexp, xs, wg_b, wu_b, wd_b)


def _dispatch_body(x2_hbm, slot_hbm, xs_hbm, idx_v, rows_v, sem):
    wid = lax.axis_index("s") * 2 + lax.axis_index("c")
    pltpu.sync_copy(slot_hbm.at[wid], idx_v)
    t0 = (wid % 16) * 128
    for j in range(8):
        pltpu.sync_copy(x2_hbm.at[pl.ds(t0 + j * 16, 16)], rows_v)
        pltpu.async_copy(rows_v, xs_hbm.at[idx_v.at[j]], sem).wait()


def _dispatch_call(x2, slot3):
    mesh = plsc.VectorSubcoreMesh(core_axis_name="c", subcore_axis_name="s")
    fn = functools.partial(
        pl.kernel,
        out_type=jax.ShapeDtypeStruct((L, H), _f32),
        mesh=mesh,
        scratch_types=[
            pltpu.VMEM((8, 16), jnp.int32),
            pltpu.VMEM((16, H), _f32),
            pltpu.SemaphoreType.DMA,
        ],
    )(_dispatch_body)
    return fn(x2, slot3)


def _combine_body(ys_hbm, slot_hbm, wts_hbm, h1_hbm, out_hbm,
                  idx0_v, idx1_v, w0_v, w1_v, rows0_v, rows1_v, h1_v, out_v,
                  sem):
    wid = lax.axis_index("s") * 2 + lax.axis_index("c")
    r0 = wid // 2
    half = 4 * (wid % 2)
    pltpu.sync_copy(slot_hbm.at[r0, pl.ds(half, 4)], idx0_v)
    pltpu.sync_copy(slot_hbm.at[16 + r0, pl.ds(half, 4)], idx1_v)
    pltpu.sync_copy(wts_hbm.at[r0, pl.ds(half, 4)], w0_v)
    pltpu.sync_copy(wts_hbm.at[16 + r0, pl.ds(half, 4)], w1_v)
    tok0 = wid * 64
    for c in range(8):
        tb = tok0 + c * 8
        pltpu.sync_copy(h1_hbm.at[pl.ds(tb, 8)], h1_v)
        i0 = idx0_v.at[c // 2, pl.ds(8 * (c % 2), 8)]
        i1 = idx1_v.at[c // 2, pl.ds(8 * (c % 2), 8)]
        pltpu.async_copy(ys_hbm.at[i0], rows0_v, sem).wait()
        pltpu.async_copy(ys_hbm.at[i1], rows1_v, sem).wait()
        w0row = w0_v[c // 2, :]
        w1row = w1_v[c // 2, :]
        for j in range(8):
            w0s = w0row[8 * (c % 2) + j]
            w1s = w1row[8 * (c % 2) + j]

            def body(i, _):
                sl = pl.ds(i * 16, 16)
                out_v[j, sl] = (h1_v[j, sl] + w0s * rows0_v[j, sl]
                                + w1s * rows1_v[j, sl])
                return 0

            lax.fori_loop(0, H // 16, body, 0)
        pltpu.sync_copy(out_v, out_hbm.at[pl.ds(tb, 8)])


def _combine_call(ys, slot3, wts3, h1):
    mesh = plsc.VectorSubcoreMesh(core_axis_name="c", subcore_axis_name="s")
    fn = functools.partial(
        pl.kernel,
        out_type=jax.ShapeDtypeStruct((S, H), _f32),
        mesh=mesh,
        scratch_types=[
            pltpu.VMEM((4, 16), jnp.int32),
            pltpu.VMEM((4, 16), jnp.int32),
            pltpu.VMEM((4, 16), _f32),
            pltpu.VMEM((4, 16), _f32),
            pltpu.VMEM((8, H), _f32),
            pltpu.VMEM((8, H), _f32),
            pltpu.VMEM((8, H), _f32),
            pltpu.VMEM((8, H), _f32),
            pltpu.SemaphoreType.DMA,
        ],
    )(_combine_body)
    return fn(ys, slot3, wts3, h1)


def kernel(hidden_states, ln1_w, ln2_w, w_qkv, w_o, q_norm_w, k_norm_w,
           router_w, expert_bias, w_gate, w_up, w_down):
    hs2 = hidden_states.reshape(S, H)
    h = _rmsnorm(hidden_states, ln1_w).reshape(S, H)
    qkv = _qkv_call(h.astype(_bf16), w_qkv.astype(_bf16))

    q = qkv[:, : NH * HD].reshape(1, S, NH, HD)
    k = qkv[:, NH * HD : (NH + NKV) * HD].reshape(1, S, NKV, HD)
    v = qkv[:, (NH + NKV) * HD :].reshape(1, S, NKV, HD)
    pos = jnp.arange(S)
    qr = _rope(_rmsnorm(q, q_norm_w), pos)[0]
    kr = _rope(_rmsnorm(k, k_norm_w), pos)[0]
    qb3 = (qr * (1.0 / 8.0)).astype(_bf16).transpose(1, 0, 2)
    kb3 = kr.astype(_bf16).transpose(1, 0, 2)
    vb3 = v[0].astype(_bf16).transpose(1, 0, 2)
    attn = _attn_call(qb3, kb3, vb3)

    h1 = _oproj_call(attn, hs2, w_o.astype(_bf16))
    x2 = _rmsnorm(h1, ln2_w)
    wr_pad = jnp.pad(router_w, ((0, 0), (0, 128 - E))).astype(_bf16)
    lg = _logits_call(x2.astype(_bf16), wr_pad)

    lg3 = lg[:, :E].reshape(16, 128, E)
    slot, wts, bexp = _router_call(lg3, expert_bias.reshape(1, 1, E))
    bexp_s = bexp.reshape(-1)[:NBLK]
    slot3 = slot.reshape(32, 8, 16)
    wts3 = wts.reshape(32, 8, 16)

    xs = _dispatch_call(x2, slot3)
    ys = _moe_call(bexp_s, xs, w_gate.astype(_bf16), w_up.astype(_bf16),
                   w_down.astype(_bf16))
    out = _combine_call(ys, slot3, wts3, h1)
    return out.reshape(1, S, H)
